# baseline (speedup 1.0000x reference)
import functools

import jax
import jax.numpy as jnp
import numpy as np
from jax import lax
from jax.experimental import pallas as pl
from jax.experimental.pallas import tpu as pltpu
from jax.experimental.pallas import tpu_sc as plsc

_GATHER_DNUMS = lax.GatherDimensionNumbers(
    offset_dims=(), collapsed_slice_dims=(0,), start_index_map=(0,))


def _allsum(v):
    lanes = lax.iota(jnp.int32, 16)
    for k in (8, 4, 2, 1):
        p = (lanes ^ k).reshape(16, 1)
        v = v + lax.gather(v, p, _GATHER_DNUMS, (1,),
                           mode=lax.GatherScatterMode.PROMISE_IN_BOUNDS)
    return v

_B, _S, _D = 4, 8192, 768
_N = _B * _S
_LN_EPS = 1e-12
_NJ = _D // 16

_NUM_WORKERS = 32
_ROWS_PER_W = _N // _NUM_WORKERS
_CH = 16
_NSTEP = _ROWS_PER_W // _CH
_TG = 8


def _sc_fused(in2d, idx, ttf, table, ttab, gamma, beta):
    mesh = plsc.VectorSubcoreMesh(core_axis_name="c", subcore_axis_name="s")

    @functools.partial(
        pl.kernel,
        out_type=jax.ShapeDtypeStruct((_N, _D), jnp.float32),
        mesh=mesh,
        scratch_types=[
            pltpu.VMEM((_ROWS_PER_W,), jnp.int32),
            pltpu.VMEM((_ROWS_PER_W,), jnp.float32),
            pltpu.VMEM((_CH, _D), jnp.float32),
            pltpu.VMEM((_CH, _D), jnp.float32),
            pltpu.VMEM((_CH, _D), jnp.float32),
            pltpu.VMEM((_CH, _D), jnp.float32),
            pltpu.VMEM((_CH, _D), jnp.float32),
            pltpu.VMEM((_CH, _D), jnp.float32),
            pltpu.VMEM((_D,), jnp.float32),
            pltpu.VMEM((_D,), jnp.float32),
            pltpu.VMEM((1, _D), jnp.float32),
            pltpu.VMEM((1, _D), jnp.float32),
            pltpu.SemaphoreType.DMA,
            pltpu.SemaphoreType.DMA,
            pltpu.SemaphoreType.DMA,
            pltpu.SemaphoreType.DMA,
            pltpu.SemaphoreType.DMA,
            pltpu.SemaphoreType.DMA,
        ],
    )
    def k(in_hbm, idx_hbm, ttf_hbm, tab_hbm, ttab_hbm, gam_hbm, bet_hbm,
          out_hbm, idx_v, ttf_v, x0, x1, p0, p1, o0, o1, g_v, b_v, r0_v, d_v,
          sin0, sin1, sp0, sp1, so0, so1):
        nc = plsc.get_sparse_core_info().num_cores
        wid = lax.axis_index("s") * nc + lax.axis_index("c")
        gbase = wid * _ROWS_PER_W
        xb = (x0, x1)
        pb = (p0, p1)
        ob = (o0, o1)
        sin = (sin0, sin1)
        sp = (sp0, sp1)
        so = (so0, so1)

        pltpu.sync_copy(idx_hbm.at[pl.ds(gbase, _ROWS_PER_W)], idx_v)
        pltpu.sync_copy(ttf_hbm.at[pl.ds(gbase, _ROWS_PER_W)], ttf_v)
        pltpu.sync_copy(gam_hbm, g_v)
        pltpu.sync_copy(bet_hbm, b_v)
        pltpu.sync_copy(ttab_hbm.at[pl.ds(0, 1)], r0_v)
        pltpu.sync_copy(ttab_hbm.at[pl.ds(1, 1)], d_v)
        for j in range(_NJ):
            sl = pl.ds(j * 16, 16)
            d_v[0, sl] = d_v[0, sl] - r0_v[0, sl]

        def start_in(c, b):
            pltpu.async_copy(in_hbm.at[pl.ds(gbase + c * _CH, _CH)], xb[b], sin[b])
            pltpu.async_copy(tab_hbm.at[idx_v.at[pl.ds(c * _CH, _CH)]], pb[b], sp[b])

        def wait_in(b):
            pltpu.make_async_copy(
                in_hbm.at[pl.ds(0, _CH)], xb[b], sin[b]).wait()
            pltpu.make_async_copy(
                tab_hbm.at[idx_v.at[pl.ds(0, _CH)]], pb[b], sp[b]).wait()

        def start_out(c, b):
            pltpu.async_copy(ob[b], out_hbm.at[pl.ds(gbase + c * _CH, _CH)], so[b])

        def wait_out(b):
            pltpu.make_async_copy(
                ob[b], out_hbm.at[pl.ds(0, _CH)], so[b]).wait()

        def compute_chunk(c, b):
            tbase = c * _CH
            tv = ttf_v[pl.ds(tbase, 16)]
            for tg in range(_CH // _TG):
                t0 = tg * _TG
                fb = [jnp.full((16,), tv[t0 + t], jnp.float32)
                      for t in range(_TG)]

                def jbody(j, accs):
                    sl = pl.ds(j * 16, 16)
                    r0j = r0_v[0, sl]
                    dj = d_v[0, sl]
                    out = []
                    for t in range(_TG):
                        v = xb[b][t0 + t, sl] + pb[b][t0 + t, sl] \
                            + (r0j + fb[t] * dj)
                        xb[b][t0 + t, sl] = v
                        out.append(accs[2 * t] + v)
                        out.append(accs[2 * t + 1] + v * v)
                    return tuple(out)

                zero = jnp.zeros((16,), jnp.float32)
                accs = lax.fori_loop(0, _NJ, jbody, (zero,) * (2 * _TG))

                mb, rb = [], []
                magic = jnp.full((16,), 0x5F3759DF, jnp.int32)
                for t in range(_TG):
                    s = _allsum(accs[2 * t])
                    q = _allsum(accs[2 * t + 1])
                    m = s * (1.0 / _D)
                    var = q * (1.0 / _D) - m * m + _LN_EPS
                    i = lax.bitcast_convert_type(var, jnp.int32)
                    y = lax.bitcast_convert_type(magic - (i >> 1), jnp.float32)
                    for _ in range(3):
                        y = y * (1.5 - 0.5 * var * y * y)
                    mb.append(m)
                    rb.append(y)

                def jbody2(j, carry):
                    sl = pl.ds(j * 16, 16)
                    gj = g_v[sl]
                    bj = b_v[sl]
                    for t in range(_TG):
                        x = xb[b][t0 + t, sl]
                        ob[b][t0 + t, sl] = (x - mb[t]) * rb[t] * gj + bj
                    return carry

                lax.fori_loop(0, _NJ, jbody2, 0)

        def process(c, b, first):
            wait_in(b)
            if not first:
                wait_out(b)
            compute_chunk(c, b)
            start_out(c, b)
            cc = jnp.minimum(c + 2, _NSTEP - 1)
            start_in(cc, b)

        start_in(0, 0)
        start_in(1, 1)
        process(0, 0, True)
        process(1, 1, True)

        def body(g):
            for b in range(2):
                process(g + b, b, False)

        pl.loop(2, _NSTEP, step=2)(body)
        for b in range(2):
            wait_in(b)
            wait_out(b)

    return k(in2d, idx, ttf, table, ttab, gamma, beta)


@jax.jit
def kernel(inputs_embeds, position_ids, token_type_ids, pos_table, type_table,
           ln_gamma, ln_beta):
    idx = position_ids.reshape(_N)
    ttf = token_type_ids.reshape(_N).astype(jnp.float32)
    inputs2d = inputs_embeds.reshape(_N, _D)
    out2d = _sc_fused(inputs2d, idx, ttf, pos_table, type_table,
                      ln_gamma, ln_beta)
    return out2d.reshape(_B, _S, _D)
